# Initial kernel scaffold; baseline (speedup 1.0000x reference)
import jax
import jax.numpy as jnp
from jax.experimental import pallas as pl


def kernel(x_ncl, w_kcd, b_row):
    raise NotImplementedError("write your pallas kernel here")


# speedup vs baseline: 3.9139x; 3.9139x over previous
import functools

import jax
import jax.numpy as jnp
from jax import lax
from jax.experimental import pallas as pl
from jax.experimental.pallas import tpu as pltpu


def _conv_kernel(x_ref, w_ref, b_ref, o_ref, *, K, TL):
    win = x_ref[0]
    taps = jnp.concatenate([win[:, k:k + TL] for k in range(K)], axis=0)
    acc = lax.dot_general(
        taps, w_ref[...],
        dimension_numbers=(((0,), (0,)), ((), ())),
        preferred_element_type=jnp.float32)
    o_ref[0] = jnp.maximum(acc + b_ref[...], 0.0).astype(o_ref.dtype)


def kernel(x_ncl, w_kcd, b_row):
    B, C, L = x_ncl.shape
    K, _, D = w_kcd.shape
    pad = (K - 1) // 2
    halo = K - 1
    TL = L
    xp = jnp.pad(x_ncl, ((0, 0), (0, 0), (pad, pad)))
    w_flat = w_kcd.reshape(K * C, D)
    body = functools.partial(_conv_kernel, K=K, TL=TL)
    return pl.pallas_call(
        body,
        out_shape=jax.ShapeDtypeStruct((B, L, D), jnp.float32),
        grid=(B,),
        in_specs=[
            pl.BlockSpec((1, C, TL + halo), lambda b: (b, 0, 0)),
            pl.BlockSpec((K * C, D), lambda b: (0, 0)),
            pl.BlockSpec((1, D), lambda b: (0, 0)),
        ],
        out_specs=pl.BlockSpec((1, TL, D), lambda b: (b, 0, 0)),
        compiler_params=pltpu.CompilerParams(
            dimension_semantics=("parallel",),
            vmem_limit_bytes=64 * 1024 * 1024,
        ),
    )(xp, w_flat, b_row)
